# baseline (speedup 1.0000x reference)
import functools

import jax
import jax.numpy as jnp
from jax import lax
from jax.experimental import pallas as pl
from jax.experimental.pallas import tpu as pltpu
from jax.experimental.pallas import tpu_sc as plsc

_B, _N, _D = 4, 8192, 768
_NW = 32
_ROWS_PER_W = _N // _NW
_R = 16
_CH = _R * _D
_STEPS = _ROWS_PER_W // _R
_ITEMS = _STEPS * _B


def _sc_body(enc_hbm, tab_hbm, out_hbm,
             tab_v, enc_v0, enc_v1, out_v0, out_v1,
             enc_s0, enc_s1, out_s0, out_s1):
    wid = lax.axis_index("s") * 2 + lax.axis_index("c")
    wbase = wid * _ROWS_PER_W
    enc_v = (enc_v0, enc_v1)
    out_v = (out_v0, out_v1)
    enc_s = (enc_s0, enc_s1)
    out_s = (out_s0, out_s1)

    def enc_off(k):
        t = lax.shift_right_logical(k, 2)
        b = lax.bitwise_and(k, 3)
        return b * (_N * _D) + (wbase + t * _R) * _D

    for p in range(2):
        pltpu.make_async_copy(
            enc_hbm.at[pl.ds(enc_off(p), _CH)], enc_v[p], enc_s[p]).start()

    def pair(g, carry):
        t = lax.shift_right_logical(g, 1)

        for p in range(2):
            k = 2 * g + p
            b = 2 * lax.bitwise_and(g, 1) + p
            off = b * (_N * _D) + (wbase + t * _R) * _D

            if p == 0:
                @pl.when(lax.bitwise_and(g, 1) == 0)
                def _():
                    pltpu.sync_copy(
                        tab_hbm.at[pl.ds((wbase + t * _R) * _D, _CH)], tab_v)

            pltpu.make_async_copy(
                enc_hbm.at[pl.ds(off, _CH)], enc_v[p], enc_s[p]).wait()

            @pl.when(g >= 1)
            def _():
                pltpu.make_async_copy(
                    out_v[p], out_hbm.at[pl.ds(off, _CH)], out_s[p]).wait()

            ev, ov = enc_v[p], out_v[p]

            @plsc.parallel_loop(0, _CH // 16, unroll=8)
            def _(j):
                sl = pl.ds(j * 16, 16)
                ov[sl] = ev[sl] + tab_v[sl]

            pltpu.make_async_copy(
                out_v[p], out_hbm.at[pl.ds(off, _CH)], out_s[p]).start()

            @pl.when(g < (_ITEMS // 2 - 1))
            def _():
                pltpu.make_async_copy(
                    enc_hbm.at[pl.ds(enc_off(k + 2), _CH)],
                    enc_v[p], enc_s[p]).start()

        return carry

    lax.fori_loop(0, _ITEMS // 2, pair, 0)

    for p in range(2):
        pltpu.make_async_copy(
            out_v[p], out_hbm.at[pl.ds(0, _CH)], out_s[p]).wait()


_sc_kernel = functools.partial(
    pl.kernel,
    mesh=plsc.VectorSubcoreMesh(core_axis_name="c", subcore_axis_name="s"),
    out_type=jax.ShapeDtypeStruct((_B * _N * _D,), jnp.float32),
    scratch_types=[
        pltpu.VMEM((_CH,), jnp.float32),
        pltpu.VMEM((_CH,), jnp.float32),
        pltpu.VMEM((_CH,), jnp.float32),
        pltpu.VMEM((_CH,), jnp.float32),
        pltpu.VMEM((_CH,), jnp.float32),
        pltpu.SemaphoreType.DMA,
        pltpu.SemaphoreType.DMA,
        pltpu.SemaphoreType.DMA,
        pltpu.SemaphoreType.DMA,
    ],
)(_sc_body)


def kernel(encoded_tokens, pos_table):
    b, n, d = encoded_tokens.shape
    flat = _sc_kernel(encoded_tokens.reshape(-1), pos_table.reshape(-1))
    return flat.reshape(b, n, d)


# speedup vs baseline: 1.0254x; 1.0254x over previous
import functools

import jax
import jax.numpy as jnp
from jax import lax
from jax.experimental import pallas as pl
from jax.experimental.pallas import tpu as pltpu
from jax.experimental.pallas import tpu_sc as plsc

_B, _N, _D = 4, 8192, 768
_NW = 32
_ROWS_PER_W = _N // _NW
_R = 32
_CH = _R * _D
_STEPS = _ROWS_PER_W // _R
_ITEMS = _STEPS * _B


def _sc_body(enc_hbm, tab_hbm, out_hbm,
             tab_v, enc_v0, enc_v1, out_v0, out_v1,
             enc_s0, enc_s1, out_s0, out_s1):
    wid = lax.axis_index("s") * 2 + lax.axis_index("c")
    wbase = wid * _ROWS_PER_W
    enc_v = (enc_v0, enc_v1)
    out_v = (out_v0, out_v1)
    enc_s = (enc_s0, enc_s1)
    out_s = (out_s0, out_s1)

    def enc_off(k):
        t = lax.shift_right_logical(k, 2)
        b = lax.bitwise_and(k, 3)
        return b * (_N * _D) + (wbase + t * _R) * _D

    for p in range(2):
        pltpu.make_async_copy(
            enc_hbm.at[pl.ds(enc_off(p), _CH)], enc_v[p], enc_s[p]).start()

    def pair(g, carry):
        t = lax.shift_right_logical(g, 1)

        for p in range(2):
            k = 2 * g + p
            b = 2 * lax.bitwise_and(g, 1) + p
            off = b * (_N * _D) + (wbase + t * _R) * _D

            if p == 0:
                @pl.when(lax.bitwise_and(g, 1) == 0)
                def _():
                    pltpu.sync_copy(
                        tab_hbm.at[pl.ds((wbase + t * _R) * _D, _CH)], tab_v)

            pltpu.make_async_copy(
                enc_hbm.at[pl.ds(off, _CH)], enc_v[p], enc_s[p]).wait()

            @pl.when(g >= 1)
            def _():
                pltpu.make_async_copy(
                    out_v[p], out_hbm.at[pl.ds(off, _CH)], out_s[p]).wait()

            ev, ov = enc_v[p], out_v[p]

            @plsc.parallel_loop(0, _CH // 16, unroll=8)
            def _(j):
                sl = pl.ds(j * 16, 16)
                ov[sl] = ev[sl] + tab_v[sl]

            pltpu.make_async_copy(
                out_v[p], out_hbm.at[pl.ds(off, _CH)], out_s[p]).start()

            @pl.when(g < (_ITEMS // 2 - 1))
            def _():
                pltpu.make_async_copy(
                    enc_hbm.at[pl.ds(enc_off(k + 2), _CH)],
                    enc_v[p], enc_s[p]).start()

        return carry

    lax.fori_loop(0, _ITEMS // 2, pair, 0)

    for p in range(2):
        pltpu.make_async_copy(
            out_v[p], out_hbm.at[pl.ds(0, _CH)], out_s[p]).wait()


_sc_kernel = functools.partial(
    pl.kernel,
    mesh=plsc.VectorSubcoreMesh(core_axis_name="c", subcore_axis_name="s"),
    out_type=jax.ShapeDtypeStruct((_B * _N * _D,), jnp.float32),
    scratch_types=[
        pltpu.VMEM((_CH,), jnp.float32),
        pltpu.VMEM((_CH,), jnp.float32),
        pltpu.VMEM((_CH,), jnp.float32),
        pltpu.VMEM((_CH,), jnp.float32),
        pltpu.VMEM((_CH,), jnp.float32),
        pltpu.SemaphoreType.DMA,
        pltpu.SemaphoreType.DMA,
        pltpu.SemaphoreType.DMA,
        pltpu.SemaphoreType.DMA,
    ],
)(_sc_body)


def kernel(encoded_tokens, pos_table):
    b, n, d = encoded_tokens.shape
    flat = _sc_kernel(encoded_tokens.reshape(-1), pos_table.reshape(-1))
    return flat.reshape(b, n, d)
